# baseline (speedup 1.0000x reference)
import functools

import jax
import jax.numpy as jnp
from jax import lax
from jax.experimental import pallas as pl
from jax.experimental.pallas import tpu as pltpu
from jax.experimental.pallas import tpu_sc as plsc

B = 16384
T = 1284
W = 2576
NC = 2
NS = 16
NW = NC * NS
BPW = B // NW
L = 16
NKEY = 1656
WROW = 1288
KPT = 52
P1 = 8
C = 16
H = BPW // C


def _annual_table_tc(qx, kx):
    def body(qx_ref, kx_ref, o_ref):
        o_ref[...] = (jnp.power(qx_ref[...] + 1.0, 1.0 / 12.0) - 1.0) * (
            1.0 - kx_ref[...]
        )

    return pl.pallas_call(
        body,
        out_shape=jax.ShapeDtypeStruct(qx.shape, jnp.float32),
    )(qx, kx)


def _make_sc_kernel():
    mesh = plsc.VectorSubcoreMesh(core_axis_name="c", subcore_axis_name="s")

    @functools.partial(
        pl.kernel,
        out_type=(
            jax.ShapeDtypeStruct((B, WROW), jnp.float32),
            jax.ShapeDtypeStruct((NC, NKEY, WROW), jnp.float32),
        ),
        mesh=mesh,
        compiler_params=pltpu.CompilerParams(use_tc_tiling_on_sc=False),
        scratch_types=[
            pltpu.VMEM((16 * W,), jnp.float32),
            pltpu.VMEM((BPW,), jnp.int32),
            pltpu.VMEM((BPW,), jnp.int32),
            pltpu.VMEM((BPW,), jnp.int32),
            pltpu.VMEM((BPW,), jnp.int32),
            pltpu.VMEM((2, C, WROW), jnp.float32),
        ]
        + [pltpu.SemaphoreType.DMA] * P1,
    )
    def sc_kern(
        t8_hbm, sex_hbm, age_hbm, dur_hbm, out_hbm, w2_hbm,
        table_v, sex_v, age_v, dur_v, key_v, chunk_v, *sems,
    ):
        cidx = lax.axis_index("c")
        wid = lax.axis_index("s") * NC + cidx
        base = wid * BPW

        pltpu.sync_copy(sex_hbm.at[pl.ds(base, BPW)], sex_v)
        pltpu.sync_copy(age_hbm.at[pl.ds(base, BPW)], age_v)
        pltpu.sync_copy(dur_hbm.at[pl.ds(base, BPW)], dur_v)
        pltpu.sync_copy(t8_hbm, table_v)

        for g in range(BPW // L):
            sl = pl.ds(g * L, L)
            mth = age_v[sl] * 12 + dur_v[sl]
            key_v[sl] = mth * 2 + sex_v[sl]

        for i in range(KPT + P1):
            if i < KPT:
                k = wid * KPT + i

                @pl.when(k < NKEY)
                def _():
                    mth = lax.shift_right_logical(k, 1)
                    sex = jnp.bitwise_and(k, 1)
                    p = jnp.bitwise_and(mth, 7)
                    start = (p * 2 + sex) * W + (mth - p)
                    start = pl.multiple_of(start, 8)
                    pltpu.make_async_copy(
                        table_v.at[pl.ds(start, WROW)],
                        w2_hbm.at[cidx, k],
                        sems[i % P1],
                    ).start()

            if i >= P1:
                i0 = i - P1
                k0 = wid * KPT + i0

                @pl.when(k0 < NKEY)
                def _():
                    pltpu.make_async_copy(
                        table_v.at[pl.ds(0, WROW)],
                        w2_hbm.at[cidx, 0],
                        sems[i0 % P1],
                    ).wait()

        plsc.subcore_barrier()

        gsem = sems[0:2]
        osem = sems[2:4]

        def gstart(d, h):
            pltpu.make_async_copy(
                w2_hbm.at[cidx].at[key_v.at[pl.ds(h * C, C)]],
                chunk_v.at[d],
                gsem[d],
            ).start()

        def gwait(d):
            pltpu.make_async_copy(
                w2_hbm.at[cidx].at[key_v.at[pl.ds(0, C)]],
                chunk_v.at[d],
                gsem[d],
            ).wait()

        def ostart(d, h):
            pltpu.make_async_copy(
                chunk_v.at[d],
                out_hbm.at[pl.ds(base + h * C, C)],
                osem[d],
            ).start()

        def owait(d):
            pltpu.make_async_copy(
                chunk_v.at[d],
                out_hbm.at[pl.ds(base, C)],
                osem[d],
            ).wait()

        gstart(0, 0)

        def body(hh, carry):
            gwait(0)
            ostart(0, hh * 2)

            @pl.when(hh >= 1)
            def _():
                owait(1)

            gstart(1, hh * 2 + 1)

            gwait(1)
            ostart(1, hh * 2 + 1)

            @pl.when(hh <= H // 2 - 2)
            def _():
                owait(0)
                gstart(0, hh * 2 + 2)

            return carry

        lax.fori_loop(0, H // 2, body, 0)
        owait(0)
        owait(1)

    return sc_kern


_SC_KERN = _make_sc_kernel()


def kernel(mp_idx, mp_val, qx, kx):
    q_ann = _annual_table_tc(qx, kx)
    q_mth = jnp.repeat(q_ann, 12, axis=1)
    t_pad = jnp.zeros((2, W + 8), jnp.float32).at[:, :T].set(q_mth)
    t8 = jnp.stack([t_pad[:, p : p + W] for p in range(8)])
    t8_flat = t8.reshape(16 * W)
    out, _ = _SC_KERN(t8_flat, mp_idx[:, 0], mp_idx[:, 1], mp_idx[:, 4])
    return out[:, :T]
